# baseline (speedup 1.0000x reference)
import functools

import jax
import jax.numpy as jnp
from jax import lax
from jax.experimental import pallas as pl
from jax.experimental.pallas import tpu as pltpu
from jax.experimental.pallas import tpu_sc as plsc

N = 10000
E = 320000
NPAD = 10112
CHUNK = 128
NW = 32
CH_PER_W = 80
EP = CHUNK * NW * CH_PER_W
RPT = NPAD // 16


def _fill_vmem_2d(ref, nrows, ncols, value):
    v = jnp.full((16,), value, jnp.float32)

    def body(r, _):
        for j in range(ncols // 16):
            ref[r, pl.ds(j * 16, 16)] = v
        return 0

    lax.fori_loop(0, nrows, body, 0)


def _copy_rows(src_ref, dst_ref, r0):
    for p in range(RPT // CHUNK):
        pltpu.sync_copy(src_ref, dst_ref.at[pl.ds(r0 + p * CHUNK, CHUNK), :])
    rem = RPT % CHUNK
    if rem:
        pltpu.sync_copy(
            src_ref.at[pl.ds(0, rem), :],
            dst_ref.at[pl.ds(r0 + (RPT // CHUNK) * CHUNK, rem), :],
        )


def _sc_degree(dst3):
    mesh = plsc.VectorSubcoreMesh(core_axis_name="c", subcore_axis_name="s", num_cores=2, num_subcores=16)
    nbuf = 4

    @functools.partial(
        pl.kernel,
        out_type=jax.ShapeDtypeStruct((2, NPAD, 16), jnp.float32),
        mesh=mesh,
        compiler_params=pltpu.CompilerParams(use_tc_tiling_on_sc=False),
        scratch_types=(
            [pltpu.VMEM((CHUNK,), jnp.int32)] * nbuf
            + [pltpu.VMEM((CHUNK, 16), jnp.float32)]
            + [pltpu.VMEM_SHARED((NPAD, 16), jnp.float32)]
            + [pltpu.SemaphoreType.DMA] * (2 * nbuf)
        ),
    )
    def k(dst_hbm, out_hbm, *scr):
        didx = scr[:nbuf]
        buf = scr[nbuf]
        acc = scr[nbuf + 1]
        dsem = scr[nbuf + 2:nbuf + 2 + nbuf]
        ssem = scr[nbuf + 2 + nbuf:]
        c = lax.axis_index("c")
        s = lax.axis_index("s")
        wid = c * 16 + s
        r0 = s * RPT

        _fill_vmem_2d(buf, CHUNK, 16, 0.0)
        _copy_rows(buf, acc, r0)
        _fill_vmem_2d(buf, CHUNK, 16, 1.0)
        plsc.subcore_barrier()

        def body(g, _):
            for b in range(nbuf):
                j = g * nbuf + b

                @pl.when((j >= nbuf) & (j < CH_PER_W + nbuf))
                def _retire():
                    pltpu.make_async_copy(buf, acc.at[didx[b]],
                                          ssem[b]).wait()

                @pl.when(j < CH_PER_W)
                def _load():
                    pltpu.async_copy(dst_hbm.at[wid, j], didx[b], dsem[b])

                bd = (b + 1) % nbuf
                jd = j - (nbuf - 1)

                @pl.when((jd >= 0) & (jd < CH_PER_W))
                def _scatter():
                    pltpu.make_async_copy(dst_hbm.at[wid, 0], didx[bd],
                                          dsem[bd]).wait()
                    pltpu.async_copy(buf, acc.at[didx[bd]], ssem[bd],
                                     add=True)
            return 0

        lax.fori_loop(0, CH_PER_W // nbuf + 1, body, 0)
        plsc.subcore_barrier()
        pltpu.sync_copy(acc.at[pl.ds(r0, RPT), :],
                        out_hbm.at[c, pl.ds(r0, RPT), :])

    return k(dst3)


def _sc_propagate(table, src3, dst3, d, nbuf):
    mesh = plsc.VectorSubcoreMesh(core_axis_name="c", subcore_axis_name="s", num_cores=2, num_subcores=16)
    assert CH_PER_W % nbuf == 0

    @functools.partial(
        pl.kernel,
        out_type=jax.ShapeDtypeStruct((2, NPAD, d), jnp.float32),
        mesh=mesh,
        compiler_params=pltpu.CompilerParams(use_tc_tiling_on_sc=False),
        scratch_types=(
            [pltpu.VMEM((CH_PER_W, CHUNK), jnp.int32)]
            + [pltpu.VMEM((CHUNK,), jnp.int32)] * nbuf
            + [pltpu.VMEM((CHUNK, d), jnp.float32)] * nbuf
            + [pltpu.VMEM_SHARED((NPAD, d), jnp.float32)]
            + [pltpu.SemaphoreType.DMA] * (3 * nbuf)
        ),
    )
    def k(tab_hbm, src_hbm, dst_hbm, out_hbm, sidx, *scr):
        didx = scr[:nbuf]
        rows = scr[nbuf:2 * nbuf]
        acc = scr[2 * nbuf]
        dsem = scr[2 * nbuf + 1:3 * nbuf + 1]
        gsem = scr[3 * nbuf + 1:4 * nbuf + 1]
        ssem = scr[4 * nbuf + 1:]
        c = lax.axis_index("c")
        s = lax.axis_index("s")
        wid = c * 16 + s
        r0 = s * RPT

        pltpu.sync_copy(src_hbm.at[wid], sidx)
        _fill_vmem_2d(rows[0], CHUNK, d, 0.0)
        _copy_rows(rows[0], acc, r0)
        plsc.subcore_barrier()

        def body(g, _):
            for b in range(nbuf):
                j = g * nbuf + b

                @pl.when((j >= nbuf) & (j < CH_PER_W + nbuf))
                def _retire():
                    pltpu.make_async_copy(rows[b], acc.at[didx[b]],
                                          ssem[b]).wait()

                @pl.when(j < CH_PER_W)
                def _load():
                    pltpu.async_copy(dst_hbm.at[wid, j], didx[b], dsem[b])
                    pltpu.async_copy(tab_hbm.at[sidx.at[j]], rows[b],
                                     gsem[b])

                bd = (b + 1) % nbuf
                jd = j - (nbuf - 1)

                @pl.when((jd >= 0) & (jd < CH_PER_W))
                def _scatter():
                    pltpu.make_async_copy(dst_hbm.at[wid, 0], didx[bd],
                                          dsem[bd]).wait()
                    pltpu.make_async_copy(tab_hbm.at[sidx.at[0]],
                                          rows[bd], gsem[bd]).wait()
                    pltpu.async_copy(rows[bd], acc.at[didx[bd]],
                                     ssem[bd], add=True)
            return 0

        lax.fori_loop(0, CH_PER_W // nbuf + 1, body, 0)
        plsc.subcore_barrier()
        pltpu.sync_copy(acc.at[pl.ds(r0, RPT), :],
                        out_hbm.at[c, pl.ds(r0, RPT), :])

    return k(table, src3, dst3)


def _tc_layer1(x, w0, degp):
    def body(x_ref, w0_ref, degp_ref, hs_ref, dinv_ref):
        deg = degp_ref[0, :N, 0:1] + degp_ref[1, :N, 0:1] + 1.0
        dinv = lax.rsqrt(deg)
        dinv_b = jnp.broadcast_to(dinv, (N, 128))
        dinv_ref[...] = dinv_b
        h = jnp.dot(x_ref[...], w0_ref[...],
                    preferred_element_type=jnp.float32)
        hs_ref[...] = h * dinv_b

    return pl.pallas_call(
        body,
        out_shape=(
            jax.ShapeDtypeStruct((N, 128), jnp.float32),
            jax.ShapeDtypeStruct((N, 128), jnp.float32),
        ),
    )(x, w0, degp)


def _tc_layer2(part1, hs, dinv_b, w1):
    def body(p_ref, hs_ref, dinv_ref, w1_ref, out_ref):
        acc = p_ref[0, :N, :] + p_ref[1, :N, :] + hs_ref[...]
        h1 = jnp.maximum(dinv_ref[...] * acc, 0.0)
        h2 = jnp.dot(h1, w1_ref[...], preferred_element_type=jnp.float32)
        out_ref[...] = h2 * dinv_ref[:, :64]

    return pl.pallas_call(
        body,
        out_shape=jax.ShapeDtypeStruct((N, 64), jnp.float32),
    )(part1, hs, dinv_b, w1)


def _tc_final(part2, hs2, dinv_b):
    def body(p_ref, hs2_ref, dinv_ref, out_ref):
        acc = p_ref[0, :N, :] + p_ref[1, :N, :] + hs2_ref[...]
        out_ref[...] = dinv_ref[:, :64] * acc

    return pl.pallas_call(
        body,
        out_shape=jax.ShapeDtypeStruct((N, 64), jnp.float32),
    )(part2, hs2, dinv_b)


def kernel(x, edge_index, W0, W1):
    src = edge_index[0].astype(jnp.int32)
    dst = edge_index[1].astype(jnp.int32)
    npad = EP - E
    pad = jnp.arange(npad, dtype=jnp.int32)
    src3 = jnp.concatenate([src, pad % N]).reshape(NW, CH_PER_W, CHUNK)
    dst3 = jnp.concatenate([dst, N + pad % (NPAD - N)]
                           ).reshape(NW, CH_PER_W, CHUNK)

    degp = _sc_degree(dst3)
    hs, dinv_b = _tc_layer1(x, W0, degp)
    part1 = _sc_propagate(hs, src3, dst3, 128, nbuf=2)
    hs2 = _tc_layer2(part1, hs, dinv_b, W1)
    part2 = _sc_propagate(hs2, src3, dst3, 64, nbuf=8)
    return _tc_final(part2, hs2, dinv_b)


# speedup vs baseline: 1.0482x; 1.0482x over previous
import functools

import jax
import jax.numpy as jnp
from jax import lax
from jax.experimental import pallas as pl
from jax.experimental.pallas import tpu as pltpu
from jax.experimental.pallas import tpu_sc as plsc

N = 10000
E = 320000
CHUNK = 128
NCH = E // CHUNK
NW = 32
RPT = N // 16

_MESH = dict(core_axis_name="c", subcore_axis_name="s",
             num_cores=2, num_subcores=16)


def _fill_vmem_2d(ref, nrows, ncols, value):
    v = jnp.full((16,), value, jnp.float32)

    def body(r, _):
        for j in range(ncols // 16):
            ref[r, pl.ds(j * 16, 16)] = v
        return 0

    lax.fori_loop(0, nrows, body, 0)


def _copy_rows(src_ref, dst_ref, r0):
    for p in range(RPT // CHUNK):
        pltpu.sync_copy(src_ref, dst_ref.at[pl.ds(r0 + p * CHUNK, CHUNK), :])
    rem = RPT % CHUNK
    if rem:
        pltpu.sync_copy(
            src_ref.at[pl.ds(0, rem), :],
            dst_ref.at[pl.ds(r0 + (RPT // CHUNK) * CHUNK, rem), :],
        )


def _nch_for(wid):
    return (NCH - wid + NW - 1) // NW


def _sc_degree(ei):
    nbuf = 4
    slag = 2

    @functools.partial(
        pl.kernel,
        out_type=jax.ShapeDtypeStruct((2, N, 16), jnp.float32),
        mesh=plsc.VectorSubcoreMesh(**_MESH),
        compiler_params=pltpu.CompilerParams(use_tc_tiling_on_sc=False),
        scratch_types=(
            [pltpu.VMEM((CHUNK,), jnp.int32)] * nbuf
            + [pltpu.VMEM((CHUNK, 16), jnp.float32)]
            + [pltpu.VMEM_SHARED((N, 16), jnp.float32)]
            + [pltpu.SemaphoreType.DMA] * (2 * nbuf)
        ),
    )
    def k(ei_hbm, out_hbm, *scr):
        didx = scr[:nbuf]
        buf = scr[nbuf]
        acc = scr[nbuf + 1]
        dsem = scr[nbuf + 2:nbuf + 2 + nbuf]
        ssem = scr[nbuf + 2 + nbuf:]
        c = lax.axis_index("c")
        s = lax.axis_index("s")
        wid = c * 16 + s
        r0 = s * RPT
        nch = _nch_for(wid)

        _fill_vmem_2d(buf, CHUNK, 16, 0.0)
        _copy_rows(buf, acc, r0)
        _fill_vmem_2d(buf, CHUNK, 16, 1.0)
        plsc.subcore_barrier()

        def body(g, _):
            for b in range(nbuf):
                j = g * nbuf + b

                @pl.when((j >= nbuf) & (j - nbuf < nch))
                def _retire():
                    pltpu.make_async_copy(buf, acc.at[didx[b]],
                                          ssem[b]).wait()

                @pl.when(j < nch)
                def _load():
                    cid = j * NW + wid
                    pltpu.async_copy(
                        ei_hbm.at[1, pl.ds(cid * CHUNK, CHUNK)],
                        didx[b], dsem[b])

                bs = (b - slag) % nbuf
                js = j - slag

                @pl.when((js >= 0) & (js < nch))
                def _scatter():
                    pltpu.make_async_copy(
                        ei_hbm.at[1, pl.ds(0, CHUNK)], didx[bs],
                        dsem[bs]).wait()
                    pltpu.async_copy(buf, acc.at[didx[bs]], ssem[bs],
                                     add=True)
            return 0

        grps = (nch + 2 * nbuf - 1) // nbuf
        lax.fori_loop(0, grps, body, 0)
        plsc.subcore_barrier()
        pltpu.sync_copy(acc.at[pl.ds(r0, RPT), :],
                        out_hbm.at[c, pl.ds(r0, RPT), :])

    return k(ei)


def _sc_propagate(table, ei, d, nbuf, slag):
    @functools.partial(
        pl.kernel,
        out_type=jax.ShapeDtypeStruct((2, N, d), jnp.float32),
        mesh=plsc.VectorSubcoreMesh(**_MESH),
        compiler_params=pltpu.CompilerParams(use_tc_tiling_on_sc=False),
        scratch_types=(
            [pltpu.VMEM((CHUNK,), jnp.int32)] * nbuf
            + [pltpu.VMEM((CHUNK,), jnp.int32)] * nbuf
            + [pltpu.VMEM((CHUNK, d), jnp.float32)] * nbuf
            + [pltpu.VMEM_SHARED((N, d), jnp.float32)]
            + [pltpu.SemaphoreType.DMA] * (4 * nbuf)
        ),
    )
    def k(tab_hbm, ei_hbm, out_hbm, *scr):
        sidx = scr[:nbuf]
        didx = scr[nbuf:2 * nbuf]
        rows = scr[2 * nbuf:3 * nbuf]
        acc = scr[3 * nbuf]
        isem = scr[3 * nbuf + 1:4 * nbuf + 1]
        dsem = scr[4 * nbuf + 1:5 * nbuf + 1]
        gsem = scr[5 * nbuf + 1:6 * nbuf + 1]
        ssem = scr[6 * nbuf + 1:]
        c = lax.axis_index("c")
        s = lax.axis_index("s")
        wid = c * 16 + s
        r0 = s * RPT
        nch = _nch_for(wid)

        _fill_vmem_2d(rows[0], CHUNK, d, 0.0)
        _copy_rows(rows[0], acc, r0)
        plsc.subcore_barrier()

        def body(g, _):
            for b in range(nbuf):
                j = g * nbuf + b

                @pl.when((j >= nbuf) & (j - nbuf < nch))
                def _retire():
                    pltpu.make_async_copy(rows[b], acc.at[didx[b]],
                                          ssem[b]).wait()

                @pl.when(j < nch)
                def _load():
                    cid = j * NW + wid
                    pltpu.async_copy(
                        ei_hbm.at[0, pl.ds(cid * CHUNK, CHUNK)],
                        sidx[b], isem[b])
                    pltpu.async_copy(
                        ei_hbm.at[1, pl.ds(cid * CHUNK, CHUNK)],
                        didx[b], dsem[b])

                bg = (b - 1) % nbuf
                jg = j - 1

                @pl.when((jg >= 0) & (jg < nch))
                def _gather():
                    pltpu.make_async_copy(
                        ei_hbm.at[0, pl.ds(0, CHUNK)], sidx[bg],
                        isem[bg]).wait()
                    pltpu.async_copy(tab_hbm.at[sidx[bg]], rows[bg],
                                     gsem[bg])

                bs = (b - slag) % nbuf
                js = j - slag

                @pl.when((js >= 0) & (js < nch))
                def _scatter():
                    pltpu.make_async_copy(
                        ei_hbm.at[1, pl.ds(0, CHUNK)], didx[bs],
                        dsem[bs]).wait()
                    pltpu.make_async_copy(tab_hbm.at[sidx[bs]], rows[bs],
                                          gsem[bs]).wait()
                    pltpu.async_copy(rows[bs], acc.at[didx[bs]],
                                     ssem[bs], add=True)
            return 0

        grps = (nch + 2 * nbuf - 1) // nbuf
        lax.fori_loop(0, grps, body, 0)
        plsc.subcore_barrier()
        pltpu.sync_copy(acc.at[pl.ds(r0, RPT), :],
                        out_hbm.at[c, pl.ds(r0, RPT), :])

    return k(table, ei)


def _tc_layer1(x, w0, degp):
    def body(x_ref, w0_ref, degp_ref, hs_ref, dinv_ref):
        deg = degp_ref[0, :, 0:1] + degp_ref[1, :, 0:1] + 1.0
        dinv = lax.rsqrt(deg)
        dinv_b = jnp.broadcast_to(dinv, (N, 128))
        dinv_ref[...] = dinv_b
        h = jnp.dot(x_ref[...], w0_ref[...],
                    preferred_element_type=jnp.float32)
        hs_ref[...] = h * dinv_b

    return pl.pallas_call(
        body,
        out_shape=(
            jax.ShapeDtypeStruct((N, 128), jnp.float32),
            jax.ShapeDtypeStruct((N, 128), jnp.float32),
        ),
    )(x, w0, degp)


def _tc_layer2(part1, hs, dinv_b, w1):
    def body(p_ref, hs_ref, dinv_ref, w1_ref, out_ref):
        acc = p_ref[0] + p_ref[1] + hs_ref[...]
        h1 = jnp.maximum(dinv_ref[...] * acc, 0.0)
        h2 = jnp.dot(h1, w1_ref[...], preferred_element_type=jnp.float32)
        out_ref[...] = h2 * dinv_ref[:, :64]

    return pl.pallas_call(
        body,
        out_shape=jax.ShapeDtypeStruct((N, 64), jnp.float32),
    )(part1, hs, dinv_b, w1)


def _tc_final(part2, hs2, dinv_b):
    def body(p_ref, hs2_ref, dinv_ref, out_ref):
        acc = p_ref[0] + p_ref[1] + hs2_ref[...]
        out_ref[...] = dinv_ref[:, :64] * acc

    return pl.pallas_call(
        body,
        out_shape=jax.ShapeDtypeStruct((N, 64), jnp.float32),
    )(part2, hs2, dinv_b)


def kernel(x, edge_index, W0, W1):
    ei = edge_index.astype(jnp.int32)
    degp = _sc_degree(ei)
    hs, dinv_b = _tc_layer1(x, W0, degp)
    part1 = _sc_propagate(hs, ei, 128, nbuf=3, slag=2)
    hs2 = _tc_layer2(part1, hs, dinv_b, W1)
    part2 = _sc_propagate(hs2, ei, 64, nbuf=4, slag=2)
    return _tc_final(part2, hs2, dinv_b)
